# baseline (speedup 1.0000x reference)
import functools

import jax
import jax.numpy as jnp
from jax import lax
from jax.experimental import pallas as pl
from jax.experimental.pallas import tpu as pltpu
from jax.experimental.pallas import tpu_sc as plsc

B = 4096
D = 128
L = 100000
K = 5

B_BLK = 4096
L_BLK = 2000
N_LBLK = L // L_BLK
DIAG_CH = 128


def _count_body(pred_ref, g_ref, lbl_ref, lab_ref, out_ref, t_ref, tm_ref):
    j = pl.program_id(1)

    @pl.when(j == 0)
    def _init():
        ii = lax.broadcasted_iota(jnp.int32, (DIAG_CH, DIAG_CH), 0)
        jj = lax.broadcasted_iota(jnp.int32, (DIAG_CH, DIAG_CH), 1)
        parts = []
        for c in range(B_BLK // DIAG_CH):
            ddc = lax.dot_general(
                pred_ref[c * DIAG_CH:(c + 1) * DIAG_CH, :],
                g_ref[c * DIAG_CH:(c + 1) * DIAG_CH, :],
                dimension_numbers=(((1,), (1,)), ((), ())),
                preferred_element_type=jnp.float32,
            )
            parts.append(jnp.sum(jnp.where(ii == jj, ddc, 0.0),
                                 axis=1, keepdims=True))
        t = jnp.concatenate(parts, axis=0)
        tb = lax.bitcast_convert_type(t, jnp.int32)
        tmb = jnp.where(t > 0, tb - 1,
                        jnp.where(t < 0, tb + 1, jnp.int32(-2147483647)))
        t_ref[...] = t
        tm_ref[...] = lax.bitcast_convert_type(tmb, jnp.float32)

    s = lax.dot_general(
        pred_ref[...], lab_ref[...],
        dimension_numbers=(((1,), (1,)), ((), ())),
        preferred_element_type=jnp.float32,
    )
    lbl = lbl_ref[...]
    col = j * L_BLK + lax.broadcasted_iota(jnp.int32, (B_BLK, L_BLK), 1)
    thr = jnp.where(col < lbl, tm_ref[...], t_ref[...])
    c = jnp.sum((s > thr).astype(jnp.float32), axis=1, keepdims=True)
    tot = jnp.where(j == 0, c, out_ref[...] + c)
    out_ref[...] = jnp.where(j == N_LBLK - 1, (tot < K).astype(jnp.float32), tot)


def _count_correct(pred, g, labels, lab_emb, interpret=False):
    return pl.pallas_call(
        _count_body,
        grid=(B // B_BLK, N_LBLK),
        in_specs=[
            pl.BlockSpec((B_BLK, D), lambda i, j: (i, 0)),
            pl.BlockSpec((B_BLK, D), lambda i, j: (i, 0)),
            pl.BlockSpec((B_BLK, 1), lambda i, j: (i, 0)),
            pl.BlockSpec((L_BLK, D), lambda i, j: (j, 0)),
        ],
        out_specs=pl.BlockSpec((B_BLK, 1), lambda i, j: (i, 0)),
        out_shape=jax.ShapeDtypeStruct((B, 1), jnp.float32),
        scratch_shapes=[
            pltpu.VMEM((B_BLK, 1), jnp.float32),
            pltpu.VMEM((B_BLK, 1), jnp.float32),
        ],
        compiler_params=pltpu.CompilerParams(
            dimension_semantics=("parallel", "arbitrary")),
        interpret=interpret,
    )(pred, g, labels.reshape(B, 1).astype(jnp.int32), lab_emb)


_INFO = plsc.get_sparse_core_info()
_NW = _INFO.num_cores * _INFO.num_subcores
_BPW = B // _NW


@functools.partial(
    pl.kernel,
    mesh=plsc.VectorSubcoreMesh(core_axis_name="c", subcore_axis_name="s"),
    out_type=jax.ShapeDtypeStruct((B, D), jnp.float32),
    scratch_types=[
        pltpu.VMEM((_BPW,), jnp.int32),
        pltpu.VMEM((_BPW, D), jnp.float32),
        pltpu.SemaphoreType.DMA,
    ],
)
def _sc_gather(table_hbm, idx_hbm, out_hbm, idx_v, rows_v, sem):
    wid = lax.axis_index("s") * _INFO.num_cores + lax.axis_index("c")
    base = wid * _BPW
    pltpu.sync_copy(idx_hbm.at[pl.ds(base, _BPW)], idx_v)
    pltpu.async_copy(table_hbm.at[idx_v], rows_v, sem).wait()
    pltpu.sync_copy(rows_v, out_hbm.at[pl.ds(base, _BPW)])


def kernel(predicted_embeddings, true_labels, label_embeddings):
    g = _sc_gather(label_embeddings, true_labels)
    correct = _count_correct(predicted_embeddings, g, true_labels,
                             label_embeddings)
    return jnp.mean(correct)


# speedup vs baseline: 1.0626x; 1.0626x over previous
import functools

import jax
import jax.numpy as jnp
from jax import lax
from jax.experimental import pallas as pl
from jax.experimental.pallas import tpu as pltpu
from jax.experimental.pallas import tpu_sc as plsc

B = 4096
D = 128
L = 100000
K = 5

B_BLK = 1024
L_BLK = 10000
N_LBLK = L // L_BLK
DIAG_CH = 128


def _count_body(pred_ref, g_ref, lbl_ref, lab_ref, out_ref, t_ref, tm_ref):
    j = pl.program_id(1)

    @pl.when(j == 0)
    def _init():
        ii = lax.broadcasted_iota(jnp.int32, (DIAG_CH, DIAG_CH), 0)
        jj = lax.broadcasted_iota(jnp.int32, (DIAG_CH, DIAG_CH), 1)
        parts = []
        for c in range(B_BLK // DIAG_CH):
            ddc = lax.dot_general(
                pred_ref[c * DIAG_CH:(c + 1) * DIAG_CH, :],
                g_ref[c * DIAG_CH:(c + 1) * DIAG_CH, :],
                dimension_numbers=(((1,), (1,)), ((), ())),
                preferred_element_type=jnp.float32,
            )
            parts.append(jnp.sum(jnp.where(ii == jj, ddc, 0.0),
                                 axis=1, keepdims=True))
        t = jnp.concatenate(parts, axis=0)
        tb = lax.bitcast_convert_type(t, jnp.int32)
        tmb = jnp.where(t > 0, tb - 1,
                        jnp.where(t < 0, tb + 1, jnp.int32(-2147483647)))
        t_ref[...] = t
        tm_ref[...] = lax.bitcast_convert_type(tmb, jnp.float32)

    s = lax.dot_general(
        pred_ref[...], lab_ref[...],
        dimension_numbers=(((1,), (1,)), ((), ())),
        preferred_element_type=jnp.float32,
    )
    lbl = lbl_ref[...]
    col = j * L_BLK + lax.broadcasted_iota(jnp.int32, (B_BLK, L_BLK), 1)
    thr = jnp.where(col < lbl, tm_ref[...], t_ref[...])
    c = jnp.sum((s > thr).astype(jnp.float32), axis=1, keepdims=True)
    tot = jnp.where(j == 0, c, out_ref[...] + c)
    out_ref[...] = jnp.where(j == N_LBLK - 1, (tot < K).astype(jnp.float32), tot)


def _count_correct(pred, g, labels, lab_emb, interpret=False):
    return pl.pallas_call(
        _count_body,
        grid=(B // B_BLK, N_LBLK),
        in_specs=[
            pl.BlockSpec((B_BLK, D), lambda i, j: (i, 0)),
            pl.BlockSpec((B_BLK, D), lambda i, j: (i, 0)),
            pl.BlockSpec((B_BLK, 1), lambda i, j: (i, 0)),
            pl.BlockSpec((L_BLK, D), lambda i, j: (j, 0)),
        ],
        out_specs=pl.BlockSpec((B_BLK, 1), lambda i, j: (i, 0)),
        out_shape=jax.ShapeDtypeStruct((B, 1), jnp.float32),
        scratch_shapes=[
            pltpu.VMEM((B_BLK, 1), jnp.float32),
            pltpu.VMEM((B_BLK, 1), jnp.float32),
        ],
        compiler_params=pltpu.CompilerParams(
            dimension_semantics=("parallel", "arbitrary")),
        interpret=interpret,
    )(pred, g, labels.reshape(B, 1).astype(jnp.int32), lab_emb)


_INFO = plsc.get_sparse_core_info()
_NW = _INFO.num_cores * _INFO.num_subcores
_BPW = B // _NW


@functools.partial(
    pl.kernel,
    mesh=plsc.VectorSubcoreMesh(core_axis_name="c", subcore_axis_name="s"),
    out_type=jax.ShapeDtypeStruct((B, D), jnp.float32),
    scratch_types=[
        pltpu.VMEM((_BPW,), jnp.int32),
        pltpu.VMEM((_BPW, D), jnp.float32),
        pltpu.SemaphoreType.DMA,
    ],
)
def _sc_gather(table_hbm, idx_hbm, out_hbm, idx_v, rows_v, sem):
    wid = lax.axis_index("s") * _INFO.num_cores + lax.axis_index("c")
    base = wid * _BPW
    pltpu.sync_copy(idx_hbm.at[pl.ds(base, _BPW)], idx_v)
    pltpu.async_copy(table_hbm.at[idx_v], rows_v, sem).wait()
    pltpu.sync_copy(rows_v, out_hbm.at[pl.ds(base, _BPW)])


def kernel(predicted_embeddings, true_labels, label_embeddings):
    g = _sc_gather(label_embeddings, true_labels)
    correct = _count_correct(predicted_embeddings, g, true_labels,
                             label_embeddings)
    return jnp.mean(correct)
